# Initial kernel scaffold; baseline (speedup 1.0000x reference)
import jax
import jax.numpy as jnp
from jax.experimental import pallas as pl


def kernel(node_features, edge_index, edge_attr, params):
    raise NotImplementedError("write your pallas kernel here")


# speedup vs baseline: 1.0470x; 1.0470x over previous
import functools

import jax
import jax.numpy as jnp
from jax import lax
from jax.experimental import pallas as pl
from jax.experimental.pallas import tpu as pltpu
from jax.experimental.pallas import tpu_sc as plsc

N = 10000
E = 160000
HID = 128
E_PAD = 163840
NW = 32
PER_W = E_PAD // NW
CHUNK = 128
NCHUNK = PER_W // CHUNK
BE = 2048
GE = E_PAD // BE
BN = 2000
GN = N // BN
N_PAD = 10240
ROWS_PER_TILE = N_PAD // 16


def _sc_mesh():
    return plsc.VectorSubcoreMesh(
        core_axis_name="c", subcore_axis_name="s", num_cores=2, num_subcores=16
    )


@functools.lru_cache(maxsize=None)
def _make_gather2(D):
    @functools.partial(
        pl.kernel,
        mesh=_sc_mesh(),
        out_type=[
            jax.ShapeDtypeStruct((E_PAD, D), jnp.float32),
            jax.ShapeDtypeStruct((E_PAD, D), jnp.float32),
        ],
        scratch_types=[
            pltpu.VMEM((CHUNK,), jnp.int32),
            pltpu.VMEM((CHUNK,), jnp.int32),
            pltpu.VMEM((CHUNK, D), jnp.float32),
            pltpu.VMEM((CHUNK, D), jnp.float32),
            pltpu.SemaphoreType.DMA,
            pltpu.SemaphoreType.DMA,
        ],
    )
    def gk(t1, t2, i1, i2, o1, o2, i1_v, i2_v, r1_v, r2_v, sem1, sem2):
        wid = lax.axis_index("s") * 2 + lax.axis_index("c")
        base = wid * PER_W

        def body(j, carry):
            off = base + j * CHUNK
            pltpu.sync_copy(i1.at[pl.ds(off, CHUNK)], i1_v)
            pltpu.sync_copy(i2.at[pl.ds(off, CHUNK)], i2_v)
            cp1 = pltpu.async_copy(t1.at[i1_v], r1_v, sem1)
            cp2 = pltpu.async_copy(t2.at[i2_v], r2_v, sem2)
            cp1.wait()
            cp2.wait()
            pltpu.sync_copy(r1_v, o1.at[pl.ds(off, CHUNK)])
            pltpu.sync_copy(r2_v, o2.at[pl.ds(off, CHUNK)])
            return carry

        lax.fori_loop(0, NCHUNK, body, 0)

    return gk


def _gather2_256(t1, t2, i1, i2):
    return _make_gather2(256)(t1, t2, i1, i2)


def _gather2_128(t1, t2, i1, i2):
    return _make_gather2(128)(t1, t2, i1, i2)


@functools.lru_cache(maxsize=None)
def _make_scatter_add():
    @functools.partial(
        pl.kernel,
        mesh=_sc_mesh(),
        out_type=jax.ShapeDtypeStruct((2, N_PAD, HID), jnp.float32),
        scratch_types=[
            pltpu.VMEM((NCHUNK, CHUNK), jnp.int32),
            pltpu.VMEM((CHUNK, HID), jnp.float32),
            pltpu.VMEM_SHARED((N_PAD, HID), jnp.float32),
            pltpu.SemaphoreType.DMA,
        ],
    )
    def sk(m, dst3, zeros, out, idx_v, rows_v, acc, sem):
        cid = lax.axis_index("c")
        sid = lax.axis_index("s")
        wid = sid * 2 + cid
        r0 = sid * ROWS_PER_TILE
        pltpu.sync_copy(zeros.at[pl.ds(r0, ROWS_PER_TILE)],
                        acc.at[pl.ds(r0, ROWS_PER_TILE)])
        pltpu.sync_copy(dst3.at[wid], idx_v)
        plsc.subcore_barrier()

        def body(j, carry):
            off = wid * PER_W + j * CHUNK
            pltpu.sync_copy(m.at[pl.ds(off, CHUNK)], rows_v)
            pltpu.sync_copy(rows_v, acc.at[idx_v.at[j]], add=True)
            return carry

        lax.fori_loop(0, NCHUNK, body, 0)
        plsc.subcore_barrier()
        pltpu.sync_copy(acc.at[pl.ds(r0, ROWS_PER_TILE)],
                        out.at[cid, pl.ds(r0, ROWS_PER_TILE)])

    return sk


def _scatter_add(m, dst3, zeros):
    return _make_scatter_add()(m, dst3, zeros)


def _dot(a, b):
    return jnp.dot(a, b, preferred_element_type=jnp.float32)


def _full(shape):
    return pl.BlockSpec(shape, lambda i: tuple(0 for _ in shape))


def _node_enc_body(nf, wne, bne, w1s, w1d, h_o, p_o, q_o):
    h = _dot(nf[...], wne[...]) + bne[...]
    h_o[...] = h
    p_o[...] = _dot(h, w1s[...])
    q_o[...] = _dot(h, w1d[...])


def _node_enc(nf, wne, bne, w1s, w1d):
    return pl.pallas_call(
        _node_enc_body,
        grid=(GN,),
        in_specs=[
            pl.BlockSpec((BN, HID), lambda i: (i, 0)),
            _full((HID, HID)), _full((1, HID)),
            _full((HID, 256)), _full((HID, 256)),
        ],
        out_specs=[
            pl.BlockSpec((BN, HID), lambda i: (i, 0)),
            pl.BlockSpec((BN, 256), lambda i: (i, 0)),
            pl.BlockSpec((BN, 256), lambda i: (i, 0)),
        ],
        out_shape=[
            jax.ShapeDtypeStruct((N, HID), jnp.float32),
            jax.ShapeDtypeStruct((N, 256), jnp.float32),
            jax.ShapeDtypeStruct((N, 256), jnp.float32),
        ],
    )(nf, wne, bne, w1s, w1d)


def _edge_enc_body(ea, wee, bee, w1e, b1, e_o, e1_o):
    e = _dot(ea[...], wee[...]) + bee[...]
    e_o[...] = e
    e1_o[...] = _dot(e, w1e[...]) + b1[...]


def _edge_enc(ea, wee, bee, w1e, b1):
    return pl.pallas_call(
        _edge_enc_body,
        grid=(GE,),
        in_specs=[
            pl.BlockSpec((BE, 16), lambda i: (i, 0)),
            _full((16, HID)), _full((1, HID)),
            _full((HID, 256)), _full((1, 256)),
        ],
        out_specs=[
            pl.BlockSpec((BE, HID), lambda i: (i, 0)),
            pl.BlockSpec((BE, 256), lambda i: (i, 0)),
        ],
        out_shape=[
            jax.ShapeDtypeStruct((E_PAD, HID), jnp.float32),
            jax.ShapeDtypeStruct((E_PAD, 256), jnp.float32),
        ],
    )(ea, wee, bee, w1e, b1)


def _msg_body(g1, g2, e1, w2, b2, m_o):
    a = jnp.maximum(g1[...] + g2[...] + e1[...], 0.0)
    m = _dot(a, w2[...]) + b2[...]
    row = pl.program_id(0) * BE + lax.broadcasted_iota(jnp.int32, (BE, HID), 0)
    m_o[...] = jnp.where(row < E, m, 0.0)


def _msg(g1, g2, e1, w2, b2):
    return pl.pallas_call(
        _msg_body,
        grid=(GE,),
        in_specs=[
            pl.BlockSpec((BE, 256), lambda i: (i, 0)),
            pl.BlockSpec((BE, 256), lambda i: (i, 0)),
            pl.BlockSpec((BE, 256), lambda i: (i, 0)),
            _full((256, HID)), _full((1, HID)),
        ],
        out_specs=pl.BlockSpec((BE, HID), lambda i: (i, 0)),
        out_shape=jax.ShapeDtypeStruct((E_PAD, HID), jnp.float32),
    )(g1, g2, e1, w2, b2)


def _make_node_upd(widths):
    no = len(widths)

    def body(*refs):
        h, agg2, wn1, wn2, bn, *rest = refs
        wrefs = rest[:no]
        h_o = rest[no]
        orefs = rest[no + 1:]
        agg = agg2[0] + agg2[1]
        hp = jnp.maximum(
            _dot(h[...], wn1[...]) + _dot(agg, wn2[...]) + bn[...], 0.0)
        h_o[...] = hp
        for w, o in zip(wrefs, orefs):
            o[...] = _dot(hp, w[...])

    def call(h, agg2, wn1, wn2, bn, ws):
        return pl.pallas_call(
            body,
            grid=(GN,),
            in_specs=[
                pl.BlockSpec((BN, HID), lambda i: (i, 0)),
                pl.BlockSpec((2, BN, HID), lambda i: (0, i, 0)),
                _full((HID, HID)), _full((HID, HID)), _full((1, HID)),
            ] + [_full((HID, w.shape[1])) for w in ws],
            out_specs=[pl.BlockSpec((BN, HID), lambda i: (i, 0))]
            + [pl.BlockSpec((BN, w.shape[1]), lambda i: (i, 0)) for w in ws],
            out_shape=[jax.ShapeDtypeStruct((N, HID), jnp.float32)]
            + [jax.ShapeDtypeStruct((N, w.shape[1]), jnp.float32) for w in ws],
        )(h, agg2, wn1, wn2, bn, *ws)

    return call


_node_upd_mid = _make_node_upd((128, 128, 256, 256))
_node_upd_last = _make_node_upd((128, 128))


def _edge_upd_body(g3, g4, e, wec, bec, w1e, b1, e_o, e1_o):
    ep = jnp.maximum(g3[...] + g4[...] + _dot(e[...], wec[...]) + bec[...], 0.0)
    e_o[...] = ep
    e1_o[...] = _dot(ep, w1e[...]) + b1[...]


def _edge_upd(g3, g4, e, wec, bec, w1e, b1):
    return pl.pallas_call(
        _edge_upd_body,
        grid=(GE,),
        in_specs=[
            pl.BlockSpec((BE, HID), lambda i: (i, 0)),
            pl.BlockSpec((BE, HID), lambda i: (i, 0)),
            pl.BlockSpec((BE, HID), lambda i: (i, 0)),
            _full((HID, HID)), _full((1, HID)),
            _full((HID, 256)), _full((1, 256)),
        ],
        out_specs=[
            pl.BlockSpec((BE, HID), lambda i: (i, 0)),
            pl.BlockSpec((BE, 256), lambda i: (i, 0)),
        ],
        out_shape=[
            jax.ShapeDtypeStruct((E_PAD, HID), jnp.float32),
            jax.ShapeDtypeStruct((E_PAD, 256), jnp.float32),
        ],
    )(g3, g4, e, wec, bec, w1e, b1)


def _cls_body(u1, u2, ea, wce, bc1, gsc, beta, wc2, bc2, wc3, bc3, o):
    x = jnp.maximum(u1[...] + u2[...] + _dot(ea[...], wce[...]) + bc1[...], 0.0)
    x = x * gsc[...] + beta[...]
    x = jnp.maximum(_dot(x, wc2[...]) + bc2[...], 0.0)
    o[...] = _dot(x, wc3[...]) + bc3[...]


def _cls(u1, u2, ea, wce, bc1, gsc, beta, wc2, bc2, wc3, bc3):
    return pl.pallas_call(
        _cls_body,
        grid=(GE,),
        in_specs=[
            pl.BlockSpec((BE, HID), lambda i: (i, 0)),
            pl.BlockSpec((BE, HID), lambda i: (i, 0)),
            pl.BlockSpec((BE, 16), lambda i: (i, 0)),
            _full((16, HID)), _full((1, HID)),
            _full((1, HID)), _full((1, HID)),
            _full((HID, 64)), _full((1, 64)),
            _full((64, 8)), _full((1, 8)),
        ],
        out_specs=pl.BlockSpec((BE, 8), lambda i: (i, 0)),
        out_shape=jax.ShapeDtypeStruct((E_PAD, 8), jnp.float32),
    )(u1, u2, ea, wce, bc1, gsc, beta, wc2, bc2, wc3, bc3)


def kernel(node_features, edge_index, edge_attr, params):
    f32 = jnp.float32
    src = jnp.pad(edge_index[0], (0, E_PAD - E))
    dst = jnp.pad(edge_index[1], (0, E_PAD - E))
    dst3 = dst.reshape(NW, NCHUNK, CHUNK)
    ea = jnp.pad(edge_attr, ((0, E_PAD - E), (0, 0)))
    zeros = jnp.zeros((N_PAD, HID), f32)

    def row(b):
        return b.reshape(1, -1)

    lyr = params["layers"]
    w1s = [lp["msg1"]["w"][:HID] for lp in lyr]
    w1d = [lp["msg1"]["w"][HID:2 * HID] for lp in lyr]
    w1e = [lp["msg1"]["w"][2 * HID:] for lp in lyr]
    b1 = [row(lp["msg1"]["b"]) for lp in lyr]
    wn1 = [lp["node_upd"]["w"][:HID] for lp in lyr]
    wn2 = [lp["node_upd"]["w"][HID:] for lp in lyr]
    bn = [row(lp["node_upd"]["b"]) for lp in lyr]
    wes = [lp["edge_upd"]["w"][:HID] for lp in lyr]
    wed = [lp["edge_upd"]["w"][HID:2 * HID] for lp in lyr]
    wec = [lp["edge_upd"]["w"][2 * HID:] for lp in lyr]
    bec = [row(lp["edge_upd"]["b"]) for lp in lyr]
    w2 = [lp["msg2"]["w"] for lp in lyr]
    b2 = [row(lp["msg2"]["b"]) for lp in lyr]
    wcs = params["cls1"]["w"][:HID]
    wcd = params["cls1"]["w"][HID:2 * HID]
    wce = params["cls1"]["w"][2 * HID:]
    bc1 = row(params["cls1"]["b"])
    inv = 1.0 / jnp.sqrt(jnp.asarray(1.0 + 1e-5, f32))
    gsc = row(params["bn_gamma"] * inv)
    beta = row(params["bn_beta"])
    wc2 = params["cls2"]["w"]
    bc2 = row(params["cls2"]["b"])
    wc3 = jnp.pad(params["cls3"]["w"], ((0, 0), (0, 6)))
    bc3 = jnp.pad(row(params["cls3"]["b"]), ((0, 0), (0, 6)))

    h, p, q = _node_enc(node_features, params["node_enc"]["w"],
                        row(params["node_enc"]["b"]), w1s[0], w1d[0])
    e, e1 = _edge_enc(ea, params["edge_enc"]["w"], row(params["edge_enc"]["b"]),
                      w1e[0], b1[0])

    for l in range(3):
        g1, g2 = _gather2_256(p, q, src, dst)
        m = _msg(g1, g2, e1, w2[l], b2[l])
        agg2 = _scatter_add(m, dst3, zeros)
        if l < 2:
            h, r, s, p, q = _node_upd_mid(
                h, agg2, wn1[l], wn2[l], bn[l],
                (wes[l], wed[l], w1s[l + 1], w1d[l + 1]))
            g3, g4 = _gather2_128(r, s, src, dst)
            e, e1 = _edge_upd(g3, g4, e, wec[l], bec[l], w1e[l + 1], b1[l + 1])
        else:
            h, t1, t2 = _node_upd_last(h, agg2, wn1[l], wn2[l], bn[l],
                                       (wcs, wcd))

    u1, u2 = _gather2_128(t1, t2, src, dst)
    logits = _cls(u1, u2, ea, wce, bc1, gsc, beta, wc2, bc2, wc3, bc3)
    return logits[:E, :2]


# speedup vs baseline: 1.1036x; 1.0541x over previous
import functools

import jax
import jax.numpy as jnp
from jax import lax
from jax.experimental import pallas as pl
from jax.experimental.pallas import tpu as pltpu
from jax.experimental.pallas import tpu_sc as plsc

N = 10000
E = 160000
HID = 128
E_PAD = 163840
NW = 32
PER_W = E_PAD // NW
CHUNK = 128
NCHUNK = PER_W // CHUNK
BE = 2048
GE = E_PAD // BE
BN = 2000
GN = N // BN
N_PAD = 10240
ROWS_PER_TILE = N_PAD // 16


def _sc_mesh():
    return plsc.VectorSubcoreMesh(
        core_axis_name="c", subcore_axis_name="s", num_cores=2, num_subcores=16
    )


@functools.lru_cache(maxsize=None)
def _make_gather2(D):
    @functools.partial(
        pl.kernel,
        mesh=_sc_mesh(),
        out_type=[
            jax.ShapeDtypeStruct((E_PAD, D), jnp.float32),
            jax.ShapeDtypeStruct((E_PAD, D), jnp.float32),
        ],
        scratch_types=[
            pltpu.VMEM((CHUNK,), jnp.int32),
            pltpu.VMEM((CHUNK,), jnp.int32),
            pltpu.VMEM((CHUNK, D), jnp.float32),
            pltpu.VMEM((CHUNK, D), jnp.float32),
            pltpu.SemaphoreType.DMA,
            pltpu.SemaphoreType.DMA,
        ],
    )
    def gk(t1, t2, i1, i2, o1, o2, i1_v, i2_v, r1_v, r2_v, sem1, sem2):
        wid = lax.axis_index("s") * 2 + lax.axis_index("c")
        base = wid * PER_W

        def body(j, carry):
            off = base + j * CHUNK
            pltpu.sync_copy(i1.at[pl.ds(off, CHUNK)], i1_v)
            pltpu.sync_copy(i2.at[pl.ds(off, CHUNK)], i2_v)
            cp1 = pltpu.async_copy(t1.at[i1_v], r1_v, sem1)
            cp2 = pltpu.async_copy(t2.at[i2_v], r2_v, sem2)
            cp1.wait()
            cp2.wait()
            pltpu.sync_copy(r1_v, o1.at[pl.ds(off, CHUNK)])
            pltpu.sync_copy(r2_v, o2.at[pl.ds(off, CHUNK)])
            return carry

        lax.fori_loop(0, NCHUNK, body, 0)

    return gk


def _gather2_256(t1, t2, i1, i2):
    return _make_gather2(256)(t1, t2, i1, i2)


def _gather2_128(t1, t2, i1, i2):
    return _make_gather2(128)(t1, t2, i1, i2)


@functools.lru_cache(maxsize=None)
def _make_scatter_add():
    @functools.partial(
        pl.kernel,
        mesh=_sc_mesh(),
        out_type=jax.ShapeDtypeStruct((2, N_PAD, HID), jnp.float32),
        scratch_types=[
            pltpu.VMEM((NCHUNK, CHUNK), jnp.int32),
            pltpu.VMEM((CHUNK, HID), jnp.float32),
            pltpu.VMEM_SHARED((N_PAD, HID), jnp.float32),
            pltpu.SemaphoreType.DMA,
        ],
    )
    def sk(m, dst3, zeros, out, idx_v, rows_v, acc, sem):
        cid = lax.axis_index("c")
        sid = lax.axis_index("s")
        wid = sid * 2 + cid
        r0 = sid * ROWS_PER_TILE
        pltpu.sync_copy(zeros.at[pl.ds(r0, ROWS_PER_TILE)],
                        acc.at[pl.ds(r0, ROWS_PER_TILE)])
        pltpu.sync_copy(dst3.at[wid], idx_v)
        plsc.subcore_barrier()

        def body(j, carry):
            off = wid * PER_W + j * CHUNK
            pltpu.sync_copy(m.at[pl.ds(off, CHUNK)], rows_v)
            pltpu.sync_copy(rows_v, acc.at[idx_v.at[j]], add=True)
            return carry

        lax.fori_loop(0, NCHUNK, body, 0)
        plsc.subcore_barrier()
        pltpu.sync_copy(acc.at[pl.ds(r0, ROWS_PER_TILE)],
                        out.at[cid, pl.ds(r0, ROWS_PER_TILE)])

    return sk


def _scatter_add(m, dst3, zeros):
    return _make_scatter_add()(m, dst3, zeros)


def _dot(a, b):
    return jnp.dot(a, b, preferred_element_type=jnp.float32)


def _full(shape):
    return pl.BlockSpec(shape, lambda i: tuple(0 for _ in shape))


def _node_enc_body(nf, wne, bne, h_o):
    h_o[...] = _dot(nf[...], wne[...]) + bne[...]


def _node_enc(nf, wne, bne):
    return pl.pallas_call(
        _node_enc_body,
        grid=(GN,),
        in_specs=[
            pl.BlockSpec((BN, HID), lambda i: (i, 0)),
            _full((HID, HID)), _full((1, HID)),
        ],
        out_specs=pl.BlockSpec((BN, HID), lambda i: (i, 0)),
        out_shape=jax.ShapeDtypeStruct((N, HID), jnp.float32),
    )(nf, wne, bne)


def _edge_enc_body(ea, wee, bee, e_o):
    e_o[...] = _dot(ea[...], wee[...]) + bee[...]


def _edge_enc(ea, wee, bee):
    return pl.pallas_call(
        _edge_enc_body,
        grid=(GE,),
        in_specs=[
            pl.BlockSpec((BE, 16), lambda i: (i, 0)),
            _full((16, HID)), _full((1, HID)),
        ],
        out_specs=pl.BlockSpec((BE, HID), lambda i: (i, 0)),
        out_shape=jax.ShapeDtypeStruct((E_PAD, HID), jnp.float32),
    )(ea, wee, bee)


def _msg_body(hs, hd, e, w1s, w1d, w1e, b1, w2, b2, m_o):
    a = (_dot(hs[...], w1s[...]) + _dot(hd[...], w1d[...])
         + _dot(e[...], w1e[...]) + b1[...])
    m = _dot(jnp.maximum(a, 0.0), w2[...]) + b2[...]
    row = pl.program_id(0) * BE + lax.broadcasted_iota(jnp.int32, (BE, HID), 0)
    m_o[...] = jnp.where(row < E, m, 0.0)


def _msg(hs, hd, e, w1s, w1d, w1e, b1, w2, b2):
    return pl.pallas_call(
        _msg_body,
        grid=(GE,),
        in_specs=[
            pl.BlockSpec((BE, HID), lambda i: (i, 0)),
            pl.BlockSpec((BE, HID), lambda i: (i, 0)),
            pl.BlockSpec((BE, HID), lambda i: (i, 0)),
            _full((HID, 256)), _full((HID, 256)), _full((HID, 256)),
            _full((1, 256)), _full((256, HID)), _full((1, HID)),
        ],
        out_specs=pl.BlockSpec((BE, HID), lambda i: (i, 0)),
        out_shape=jax.ShapeDtypeStruct((E_PAD, HID), jnp.float32),
    )(hs, hd, e, w1s, w1d, w1e, b1, w2, b2)


def _make_node_upd(widths):
    no = len(widths)

    def body(*refs):
        h, agg2, wn1, wn2, bn, *rest = refs
        wrefs = rest[:no]
        h_o = rest[no]
        orefs = rest[no + 1:]
        agg = agg2[0] + agg2[1]
        hp = jnp.maximum(
            _dot(h[...], wn1[...]) + _dot(agg, wn2[...]) + bn[...], 0.0)
        h_o[...] = hp
        for w, o in zip(wrefs, orefs):
            o[...] = _dot(hp, w[...])

    def call(h, agg2, wn1, wn2, bn, ws):
        return pl.pallas_call(
            body,
            grid=(GN,),
            in_specs=[
                pl.BlockSpec((BN, HID), lambda i: (i, 0)),
                pl.BlockSpec((2, BN, HID), lambda i: (0, i, 0)),
                _full((HID, HID)), _full((HID, HID)), _full((1, HID)),
            ] + [_full((HID, w.shape[1])) for w in ws],
            out_specs=[pl.BlockSpec((BN, HID), lambda i: (i, 0))]
            + [pl.BlockSpec((BN, w.shape[1]), lambda i: (i, 0)) for w in ws],
            out_shape=[jax.ShapeDtypeStruct((N, HID), jnp.float32)]
            + [jax.ShapeDtypeStruct((N, w.shape[1]), jnp.float32) for w in ws],
        )(h, agg2, wn1, wn2, bn, *ws)

    return call


_node_upd2 = _make_node_upd((128, 128))


def _edge_upd_body(g3, g4, e, wec, bec, e_o):
    e_o[...] = jnp.maximum(
        g3[...] + g4[...] + _dot(e[...], wec[...]) + bec[...], 0.0)


def _edge_upd(g3, g4, e, wec, bec):
    return pl.pallas_call(
        _edge_upd_body,
        grid=(GE,),
        in_specs=[
            pl.BlockSpec((BE, HID), lambda i: (i, 0)),
            pl.BlockSpec((BE, HID), lambda i: (i, 0)),
            pl.BlockSpec((BE, HID), lambda i: (i, 0)),
            _full((HID, HID)), _full((1, HID)),
        ],
        out_specs=pl.BlockSpec((BE, HID), lambda i: (i, 0)),
        out_shape=jax.ShapeDtypeStruct((E_PAD, HID), jnp.float32),
    )(g3, g4, e, wec, bec)


def _cls_body(u1, u2, ea, wce, bc1, gsc, beta, wc2, bc2, wc3, bc3, o):
    x = jnp.maximum(u1[...] + u2[...] + _dot(ea[...], wce[...]) + bc1[...], 0.0)
    x = x * gsc[...] + beta[...]
    x = jnp.maximum(_dot(x, wc2[...]) + bc2[...], 0.0)
    o[...] = _dot(x, wc3[...]) + bc3[...]


def _cls(u1, u2, ea, wce, bc1, gsc, beta, wc2, bc2, wc3, bc3):
    return pl.pallas_call(
        _cls_body,
        grid=(GE,),
        in_specs=[
            pl.BlockSpec((BE, HID), lambda i: (i, 0)),
            pl.BlockSpec((BE, HID), lambda i: (i, 0)),
            pl.BlockSpec((BE, 16), lambda i: (i, 0)),
            _full((16, HID)), _full((1, HID)),
            _full((1, HID)), _full((1, HID)),
            _full((HID, 64)), _full((1, 64)),
            _full((64, 8)), _full((1, 8)),
        ],
        out_specs=pl.BlockSpec((BE, 8), lambda i: (i, 0)),
        out_shape=jax.ShapeDtypeStruct((E_PAD, 8), jnp.float32),
    )(u1, u2, ea, wce, bc1, gsc, beta, wc2, bc2, wc3, bc3)


def kernel(node_features, edge_index, edge_attr, params):
    f32 = jnp.float32
    src = jnp.pad(edge_index[0], (0, E_PAD - E))
    dst = jnp.pad(edge_index[1], (0, E_PAD - E))
    dst3 = dst.reshape(NW, NCHUNK, CHUNK)
    ea = jnp.pad(edge_attr, ((0, E_PAD - E), (0, 0)))
    zeros = jnp.zeros((N_PAD, HID), f32)

    def row(b):
        return b.reshape(1, -1)

    lyr = params["layers"]
    w1s = [lp["msg1"]["w"][:HID] for lp in lyr]
    w1d = [lp["msg1"]["w"][HID:2 * HID] for lp in lyr]
    w1e = [lp["msg1"]["w"][2 * HID:] for lp in lyr]
    b1 = [row(lp["msg1"]["b"]) for lp in lyr]
    wn1 = [lp["node_upd"]["w"][:HID] for lp in lyr]
    wn2 = [lp["node_upd"]["w"][HID:] for lp in lyr]
    bn = [row(lp["node_upd"]["b"]) for lp in lyr]
    wes = [lp["edge_upd"]["w"][:HID] for lp in lyr]
    wed = [lp["edge_upd"]["w"][HID:2 * HID] for lp in lyr]
    wec = [lp["edge_upd"]["w"][2 * HID:] for lp in lyr]
    bec = [row(lp["edge_upd"]["b"]) for lp in lyr]
    w2 = [lp["msg2"]["w"] for lp in lyr]
    b2 = [row(lp["msg2"]["b"]) for lp in lyr]
    wcs = params["cls1"]["w"][:HID]
    wcd = params["cls1"]["w"][HID:2 * HID]
    wce = params["cls1"]["w"][2 * HID:]
    bc1 = row(params["cls1"]["b"])
    inv = 1.0 / jnp.sqrt(jnp.asarray(1.0 + 1e-5, f32))
    gsc = row(params["bn_gamma"] * inv)
    beta = row(params["bn_beta"])
    wc2 = params["cls2"]["w"]
    bc2 = row(params["cls2"]["b"])
    wc3 = jnp.pad(params["cls3"]["w"], ((0, 0), (0, 6)))
    bc3 = jnp.pad(row(params["cls3"]["b"]), ((0, 0), (0, 6)))

    h = _node_enc(node_features, params["node_enc"]["w"],
                  row(params["node_enc"]["b"]))
    e = _edge_enc(ea, params["edge_enc"]["w"], row(params["edge_enc"]["b"]))

    for l in range(3):
        hs, hd = _gather2_128(h, h, src, dst)
        m = _msg(hs, hd, e, w1s[l], w1d[l], w1e[l], b1[l], w2[l], b2[l])
        agg2 = _scatter_add(m, dst3, zeros)
        if l < 2:
            h, r, s = _node_upd2(h, agg2, wn1[l], wn2[l], bn[l],
                                 (wes[l], wed[l]))
            g3, g4 = _gather2_128(r, s, src, dst)
            e = _edge_upd(g3, g4, e, wec[l], bec[l])
        else:
            h, t1, t2 = _node_upd2(h, agg2, wn1[l], wn2[l], bn[l], (wcs, wcd))

    u1, u2 = _gather2_128(t1, t2, src, dst)
    logits = _cls(u1, u2, ea, wce, bc1, gsc, beta, wc2, bc2, wc3, bc3)
    return logits[:E, :2]


# speedup vs baseline: 1.2869x; 1.1661x over previous
import functools

import jax
import jax.numpy as jnp
from jax import lax
from jax.experimental import pallas as pl
from jax.experimental.pallas import tpu as pltpu
from jax.experimental.pallas import tpu_sc as plsc

N = 10000
E = 160000
HID = 128
E_PAD = 163840
NW = 32
PER_W = E_PAD // NW
CHUNK = 128
NCHUNK = PER_W // CHUNK
BE = 2048
GE = E_PAD // BE
BN = 2000
GN = N // BN
N_PAD = 10240
ROWS_PER_TILE = N_PAD // 16


def _sc_mesh():
    return plsc.VectorSubcoreMesh(
        core_axis_name="c", subcore_axis_name="s", num_cores=2, num_subcores=16
    )


K = 3


@functools.lru_cache(maxsize=None)
def _make_gather2(D):
    @functools.partial(
        pl.kernel,
        mesh=_sc_mesh(),
        out_type=[
            jax.ShapeDtypeStruct((E_PAD, D), jnp.float32),
            jax.ShapeDtypeStruct((E_PAD, D), jnp.float32),
        ],
        scratch_types=[
            pltpu.VMEM((NCHUNK, CHUNK), jnp.int32),
            pltpu.VMEM((NCHUNK, CHUNK), jnp.int32),
        ]
        + [pltpu.VMEM((CHUNK, D), jnp.float32) for _ in range(2 * K)]
        + [pltpu.SemaphoreType.DMA for _ in range(4 * K)],
    )
    def gk(t1, t2, i1, i2, o1, o2, i1_v, i2_v, *rest):
        r1 = rest[0:K]
        r2 = rest[K:2 * K]
        gs1 = rest[2 * K:3 * K]
        gs2 = rest[3 * K:4 * K]
        ws1 = rest[4 * K:5 * K]
        ws2 = rest[5 * K:6 * K]
        wid = lax.axis_index("s") * 2 + lax.axis_index("c")
        base = wid * PER_W
        pltpu.sync_copy(i1.at[wid], i1_v)
        pltpu.sync_copy(i2.at[wid], i2_v)
        g1 = [None] * K
        g2 = [None] * K
        w1 = [None] * K
        w2 = [None] * K
        for j in range(K):
            g1[j] = pltpu.async_copy(t1.at[i1_v.at[j]], r1[j], gs1[j])
            g2[j] = pltpu.async_copy(t2.at[i2_v.at[j]], r2[j], gs2[j])
        for j in range(NCHUNK):
            s = j % K
            off = base + j * CHUNK
            g1[s].wait()
            g2[s].wait()
            w1[s] = pltpu.async_copy(r1[s], o1.at[pl.ds(off, CHUNK)], ws1[s])
            w2[s] = pltpu.async_copy(r2[s], o2.at[pl.ds(off, CHUNK)], ws2[s])
            nj = j + K
            if nj < NCHUNK:
                w1[s].wait()
                w2[s].wait()
                g1[s] = pltpu.async_copy(t1.at[i1_v.at[nj]], r1[s], gs1[s])
                g2[s] = pltpu.async_copy(t2.at[i2_v.at[nj]], r2[s], gs2[s])
        for j in range(NCHUNK - K, NCHUNK):
            s = j % K
            w1[s].wait()
            w2[s].wait()

    return gk


def _gather2_128(t1, t2, i1, i2):
    return _make_gather2(128)(t1, t2, i1, i2)


KS = 2


@functools.lru_cache(maxsize=None)
def _make_scatter_add():
    @functools.partial(
        pl.kernel,
        mesh=_sc_mesh(),
        out_type=jax.ShapeDtypeStruct((2, N_PAD, HID), jnp.float32),
        scratch_types=[
            pltpu.VMEM((NCHUNK, CHUNK), jnp.int32),
            pltpu.VMEM_SHARED((N_PAD, HID), jnp.float32),
        ]
        + [pltpu.VMEM((CHUNK, HID), jnp.float32) for _ in range(KS)]
        + [pltpu.SemaphoreType.DMA for _ in range(2 * KS)],
    )
    def sk(m, dst3, zeros, out, idx_v, acc, *rest):
        rows = rest[0:KS]
        ls = rest[KS:2 * KS]
        ss = rest[2 * KS:3 * KS]
        cid = lax.axis_index("c")
        sid = lax.axis_index("s")
        wid = sid * 2 + cid
        base = wid * PER_W
        r0 = sid * ROWS_PER_TILE
        pltpu.sync_copy(dst3.at[wid], idx_v)
        ld = [None] * KS
        sc = [None] * KS
        for j in range(KS):
            ld[j] = pltpu.async_copy(
                m.at[pl.ds(base + j * CHUNK, CHUNK)], rows[j], ls[j])
        pltpu.sync_copy(zeros.at[pl.ds(r0, ROWS_PER_TILE)],
                        acc.at[pl.ds(r0, ROWS_PER_TILE)])
        plsc.subcore_barrier()
        for j in range(NCHUNK):
            s = j % KS
            ld[s].wait()
            sc[s] = pltpu.async_copy(rows[s], acc.at[idx_v.at[j]], ss[s],
                                     add=True)
            nj = j + KS
            if nj < NCHUNK:
                sc[s].wait()
                ld[s] = pltpu.async_copy(
                    m.at[pl.ds(base + nj * CHUNK, CHUNK)], rows[s], ls[s])
        for j in range(NCHUNK - KS, NCHUNK):
            sc[j % KS].wait()
        plsc.subcore_barrier()
        pltpu.sync_copy(acc.at[pl.ds(r0, ROWS_PER_TILE)],
                        out.at[cid, pl.ds(r0, ROWS_PER_TILE)])

    return sk


def _scatter_add(m, dst3, zeros):
    return _make_scatter_add()(m, dst3, zeros)


def _dot(a, b):
    return jnp.dot(a, b, preferred_element_type=jnp.float32)


def _full(shape):
    return pl.BlockSpec(shape, lambda i: tuple(0 for _ in shape))


def _node_enc_body(nf, wne, bne, h_o):
    h_o[...] = _dot(nf[...], wne[...]) + bne[...]


def _node_enc(nf, wne, bne):
    return pl.pallas_call(
        _node_enc_body,
        grid=(GN,),
        in_specs=[
            pl.BlockSpec((BN, HID), lambda i: (i, 0)),
            _full((HID, HID)), _full((1, HID)),
        ],
        out_specs=pl.BlockSpec((BN, HID), lambda i: (i, 0)),
        out_shape=jax.ShapeDtypeStruct((N, HID), jnp.float32),
    )(nf, wne, bne)


def _edge_enc_body(ea, wee, bee, e_o):
    e_o[...] = _dot(ea[...], wee[...]) + bee[...]


def _edge_enc(ea, wee, bee):
    return pl.pallas_call(
        _edge_enc_body,
        grid=(GE,),
        in_specs=[
            pl.BlockSpec((BE, 16), lambda i: (i, 0)),
            _full((16, HID)), _full((1, HID)),
        ],
        out_specs=pl.BlockSpec((BE, HID), lambda i: (i, 0)),
        out_shape=jax.ShapeDtypeStruct((E_PAD, HID), jnp.float32),
    )(ea, wee, bee)


def _msg_body(hs, hd, e, w1s, w1d, w1e, b1, w2, b2, m_o):
    a = (_dot(hs[...], w1s[...]) + _dot(hd[...], w1d[...])
         + _dot(e[...], w1e[...]) + b1[...])
    m = _dot(jnp.maximum(a, 0.0), w2[...]) + b2[...]
    row = pl.program_id(0) * BE + lax.broadcasted_iota(jnp.int32, (BE, HID), 0)
    m_o[...] = jnp.where(row < E, m, 0.0)


def _msg(hs, hd, e, w1s, w1d, w1e, b1, w2, b2):
    return pl.pallas_call(
        _msg_body,
        grid=(GE,),
        in_specs=[
            pl.BlockSpec((BE, HID), lambda i: (i, 0)),
            pl.BlockSpec((BE, HID), lambda i: (i, 0)),
            pl.BlockSpec((BE, HID), lambda i: (i, 0)),
            _full((HID, 256)), _full((HID, 256)), _full((HID, 256)),
            _full((1, 256)), _full((256, HID)), _full((1, HID)),
        ],
        out_specs=pl.BlockSpec((BE, HID), lambda i: (i, 0)),
        out_shape=jax.ShapeDtypeStruct((E_PAD, HID), jnp.float32),
    )(hs, hd, e, w1s, w1d, w1e, b1, w2, b2)


def _make_node_upd(widths):
    no = len(widths)

    def body(*refs):
        h, agg2, wn1, wn2, bn, *rest = refs
        wrefs = rest[:no]
        h_o = rest[no]
        orefs = rest[no + 1:]
        agg = agg2[0] + agg2[1]
        hp = jnp.maximum(
            _dot(h[...], wn1[...]) + _dot(agg, wn2[...]) + bn[...], 0.0)
        h_o[...] = hp
        for w, o in zip(wrefs, orefs):
            o[...] = _dot(hp, w[...])

    def call(h, agg2, wn1, wn2, bn, ws):
        return pl.pallas_call(
            body,
            grid=(GN,),
            in_specs=[
                pl.BlockSpec((BN, HID), lambda i: (i, 0)),
                pl.BlockSpec((2, BN, HID), lambda i: (0, i, 0)),
                _full((HID, HID)), _full((HID, HID)), _full((1, HID)),
            ] + [_full((HID, w.shape[1])) for w in ws],
            out_specs=[pl.BlockSpec((BN, HID), lambda i: (i, 0))]
            + [pl.BlockSpec((BN, w.shape[1]), lambda i: (i, 0)) for w in ws],
            out_shape=[jax.ShapeDtypeStruct((N, HID), jnp.float32)]
            + [jax.ShapeDtypeStruct((N, w.shape[1]), jnp.float32) for w in ws],
        )(h, agg2, wn1, wn2, bn, *ws)

    return call


_node_upd2 = _make_node_upd((128, 128))


def _edge_upd_body(g3, g4, e, wec, bec, e_o):
    e_o[...] = jnp.maximum(
        g3[...] + g4[...] + _dot(e[...], wec[...]) + bec[...], 0.0)


def _edge_upd(g3, g4, e, wec, bec):
    return pl.pallas_call(
        _edge_upd_body,
        grid=(GE,),
        in_specs=[
            pl.BlockSpec((BE, HID), lambda i: (i, 0)),
            pl.BlockSpec((BE, HID), lambda i: (i, 0)),
            pl.BlockSpec((BE, HID), lambda i: (i, 0)),
            _full((HID, HID)), _full((1, HID)),
        ],
        out_specs=pl.BlockSpec((BE, HID), lambda i: (i, 0)),
        out_shape=jax.ShapeDtypeStruct((E_PAD, HID), jnp.float32),
    )(g3, g4, e, wec, bec)


def _cls_body(u1, u2, ea, wce, bc1, gsc, beta, wc2, bc2, wc3, bc3, o):
    x = jnp.maximum(u1[...] + u2[...] + _dot(ea[...], wce[...]) + bc1[...], 0.0)
    x = x * gsc[...] + beta[...]
    x = jnp.maximum(_dot(x, wc2[...]) + bc2[...], 0.0)
    o[...] = _dot(x, wc3[...]) + bc3[...]


def _cls(u1, u2, ea, wce, bc1, gsc, beta, wc2, bc2, wc3, bc3):
    return pl.pallas_call(
        _cls_body,
        grid=(GE,),
        in_specs=[
            pl.BlockSpec((BE, HID), lambda i: (i, 0)),
            pl.BlockSpec((BE, HID), lambda i: (i, 0)),
            pl.BlockSpec((BE, 16), lambda i: (i, 0)),
            _full((16, HID)), _full((1, HID)),
            _full((1, HID)), _full((1, HID)),
            _full((HID, 64)), _full((1, 64)),
            _full((64, 8)), _full((1, 8)),
        ],
        out_specs=pl.BlockSpec((BE, 8), lambda i: (i, 0)),
        out_shape=jax.ShapeDtypeStruct((E_PAD, 8), jnp.float32),
    )(u1, u2, ea, wce, bc1, gsc, beta, wc2, bc2, wc3, bc3)


def kernel(node_features, edge_index, edge_attr, params):
    f32 = jnp.float32
    src3 = jnp.pad(edge_index[0], (0, E_PAD - E)).reshape(NW, NCHUNK, CHUNK)
    dst3 = jnp.pad(edge_index[1], (0, E_PAD - E)).reshape(NW, NCHUNK, CHUNK)
    ea = jnp.pad(edge_attr, ((0, E_PAD - E), (0, 0)))
    zeros = jnp.zeros((N_PAD, HID), f32)

    def row(b):
        return b.reshape(1, -1)

    lyr = params["layers"]
    w1s = [lp["msg1"]["w"][:HID] for lp in lyr]
    w1d = [lp["msg1"]["w"][HID:2 * HID] for lp in lyr]
    w1e = [lp["msg1"]["w"][2 * HID:] for lp in lyr]
    b1 = [row(lp["msg1"]["b"]) for lp in lyr]
    wn1 = [lp["node_upd"]["w"][:HID] for lp in lyr]
    wn2 = [lp["node_upd"]["w"][HID:] for lp in lyr]
    bn = [row(lp["node_upd"]["b"]) for lp in lyr]
    wes = [lp["edge_upd"]["w"][:HID] for lp in lyr]
    wed = [lp["edge_upd"]["w"][HID:2 * HID] for lp in lyr]
    wec = [lp["edge_upd"]["w"][2 * HID:] for lp in lyr]
    bec = [row(lp["edge_upd"]["b"]) for lp in lyr]
    w2 = [lp["msg2"]["w"] for lp in lyr]
    b2 = [row(lp["msg2"]["b"]) for lp in lyr]
    wcs = params["cls1"]["w"][:HID]
    wcd = params["cls1"]["w"][HID:2 * HID]
    wce = params["cls1"]["w"][2 * HID:]
    bc1 = row(params["cls1"]["b"])
    inv = 1.0 / jnp.sqrt(jnp.asarray(1.0 + 1e-5, f32))
    gsc = row(params["bn_gamma"] * inv)
    beta = row(params["bn_beta"])
    wc2 = params["cls2"]["w"]
    bc2 = row(params["cls2"]["b"])
    wc3 = jnp.pad(params["cls3"]["w"], ((0, 0), (0, 6)))
    bc3 = jnp.pad(row(params["cls3"]["b"]), ((0, 0), (0, 6)))

    h = _node_enc(node_features, params["node_enc"]["w"],
                  row(params["node_enc"]["b"]))
    e = _edge_enc(ea, params["edge_enc"]["w"], row(params["edge_enc"]["b"]))

    for l in range(3):
        hs, hd = _gather2_128(h, h, src3, dst3)
        m = _msg(hs, hd, e, w1s[l], w1d[l], w1e[l], b1[l], w2[l], b2[l])
        agg2 = _scatter_add(m, dst3, zeros)
        if l < 2:
            h, r, s = _node_upd2(h, agg2, wn1[l], wn2[l], bn[l],
                                 (wes[l], wed[l]))
            g3, g4 = _gather2_128(r, s, src3, dst3)
            e = _edge_upd(g3, g4, e, wec[l], bec[l])
        else:
            h, t1, t2 = _node_upd2(h, agg2, wn1[l], wn2[l], bn[l], (wcs, wcd))

    u1, u2 = _gather2_128(t1, t2, src3, dst3)
    logits = _cls(u1, u2, ea, wce, bc1, gsc, beta, wc2, bc2, wc3, bc3)
    return logits[:E, :2]


# speedup vs baseline: 1.5862x; 1.2326x over previous
import functools

import jax
import jax.numpy as jnp
from jax import lax
from jax.experimental import pallas as pl
from jax.experimental.pallas import tpu as pltpu
from jax.experimental.pallas import tpu_sc as plsc

N = 10000
E = 160000
HID = 128
E_PAD = 163840
NW = 32
PER_W = E_PAD // NW
CHUNK = 128
NCHUNK = PER_W // CHUNK
BE = 2048
GE = E_PAD // BE
BN = 2000
GN = N // BN
N_PAD = 10240
ROWS_PER_TILE = N_PAD // 16


def _sc_mesh():
    return plsc.VectorSubcoreMesh(
        core_axis_name="c", subcore_axis_name="s", num_cores=2, num_subcores=16
    )


@functools.lru_cache(maxsize=None)
def _make_gather2(D):
    NB = 6 if D == 128 else 3

    @functools.partial(
        pl.kernel,
        mesh=_sc_mesh(),
        out_type=[
            jax.ShapeDtypeStruct((E_PAD, D), jnp.float32),
            jax.ShapeDtypeStruct((E_PAD, D), jnp.float32),
        ],
        scratch_types=[
            pltpu.VMEM((NCHUNK, CHUNK), jnp.int32),
            pltpu.VMEM((NCHUNK, CHUNK), jnp.int32),
        ]
        + [pltpu.VMEM((CHUNK, D), jnp.float32) for _ in range(NB)]
        + [pltpu.SemaphoreType.DMA for _ in range(2 * NB)],
    )
    def gk(t1, t2, i1, i2, o1, o2, i1_v, i2_v, *rest):
        bufs = rest[0:NB]
        gsem = rest[NB:2 * NB]
        wsem = rest[2 * NB:3 * NB]
        wid = lax.axis_index("s") * 2 + lax.axis_index("c")
        base = wid * PER_W
        pltpu.sync_copy(i1.at[wid], i1_v)
        pltpu.sync_copy(i2.at[wid], i2_v)
        ops = [(t, j) for j in range(NCHUNK) for t in (0, 1)]

        def start(k, s):
            t, j = ops[k]
            idx = (i1_v if t == 0 else i2_v).at[j]
            tab = t1 if t == 0 else t2
            return pltpu.async_copy(tab.at[idx], bufs[s], gsem[s])

        def wout(k, s):
            t, j = ops[k]
            out = o1 if t == 0 else o2
            return pltpu.async_copy(
                bufs[s], out.at[pl.ds(base + j * CHUNK, CHUNK)], wsem[s])

        g = [None] * NB
        w = [None] * NB
        LAG = max(1, NB // 2)
        for k in range(len(ops)):
            s = k % NB
            if w[s] is not None:
                w[s].wait()
            g[s] = start(k, s)
            ko = k - LAG
            if ko >= 0:
                so = ko % NB
                g[so].wait()
                w[so] = wout(ko, so)
        for ko in range(len(ops) - LAG, len(ops)):
            so = ko % NB
            g[so].wait()
            w[so] = wout(ko, so)
        for ko in range(len(ops) - NB, len(ops)):
            w[ko % NB].wait()

    return gk


def _gather2_128(t1, t2, i1, i2):
    return _make_gather2(128)(t1, t2, i1, i2)


def _gather2_256(t1, t2, i1, i2):
    return _make_gather2(256)(t1, t2, i1, i2)


KS = 2


@functools.lru_cache(maxsize=None)
def _make_scatter_add():
    @functools.partial(
        pl.kernel,
        mesh=_sc_mesh(),
        out_type=jax.ShapeDtypeStruct((2, N_PAD, HID), jnp.float32),
        scratch_types=[
            pltpu.VMEM((NCHUNK, CHUNK), jnp.int32),
            pltpu.VMEM_SHARED((N_PAD, HID), jnp.float32),
        ]
        + [pltpu.VMEM((CHUNK, HID), jnp.float32) for _ in range(KS)]
        + [pltpu.SemaphoreType.DMA for _ in range(2 * KS)],
    )
    def sk(m, dst3, zeros, out, idx_v, acc, *rest):
        rows = rest[0:KS]
        ls = rest[KS:2 * KS]
        ss = rest[2 * KS:3 * KS]
        cid = lax.axis_index("c")
        sid = lax.axis_index("s")
        wid = sid * 2 + cid
        base = wid * PER_W
        r0 = sid * ROWS_PER_TILE
        pltpu.sync_copy(dst3.at[wid], idx_v)
        ld = [None] * KS
        sc = [None] * KS
        for j in range(KS):
            ld[j] = pltpu.async_copy(
                m.at[pl.ds(base + j * CHUNK, CHUNK)], rows[j], ls[j])
        pltpu.sync_copy(zeros.at[pl.ds(r0, ROWS_PER_TILE)],
                        acc.at[pl.ds(r0, ROWS_PER_TILE)])
        plsc.subcore_barrier()
        for j in range(NCHUNK):
            s = j % KS
            ld[s].wait()
            sc[s] = pltpu.async_copy(rows[s], acc.at[idx_v.at[j]], ss[s],
                                     add=True)
            nj = j + KS
            if nj < NCHUNK:
                sc[s].wait()
                ld[s] = pltpu.async_copy(
                    m.at[pl.ds(base + nj * CHUNK, CHUNK)], rows[s], ls[s])
        for j in range(NCHUNK - KS, NCHUNK):
            sc[j % KS].wait()
        plsc.subcore_barrier()
        pltpu.sync_copy(acc.at[pl.ds(r0, ROWS_PER_TILE)],
                        out.at[cid, pl.ds(r0, ROWS_PER_TILE)])

    return sk


def _scatter_add(m, dst3, zeros):
    return _make_scatter_add()(m, dst3, zeros)


def _dot(a, b):
    return jnp.dot(a, b, preferred_element_type=jnp.float32)


def _full(shape):
    return pl.BlockSpec(shape, lambda i: tuple(0 for _ in shape))


def _node_enc_body(nf, wne, bne, h_o):
    h_o[...] = _dot(nf[...], wne[...]) + bne[...]


def _node_enc(nf, wne, bne):
    return pl.pallas_call(
        _node_enc_body,
        grid=(GN,),
        in_specs=[
            pl.BlockSpec((BN, HID), lambda i: (i, 0)),
            _full((HID, HID)), _full((1, HID)),
        ],
        out_specs=pl.BlockSpec((BN, HID), lambda i: (i, 0)),
        out_shape=jax.ShapeDtypeStruct((N, HID), jnp.float32),
    )(nf, wne, bne)


def _edge_enc_body(ea, wee, bee, e_o):
    e_o[...] = _dot(ea[...], wee[...]) + bee[...]


def _edge_enc(ea, wee, bee):
    return pl.pallas_call(
        _edge_enc_body,
        grid=(GE,),
        in_specs=[
            pl.BlockSpec((BE, 16), lambda i: (i, 0)),
            _full((16, HID)), _full((1, HID)),
        ],
        out_specs=pl.BlockSpec((BE, HID), lambda i: (i, 0)),
        out_shape=jax.ShapeDtypeStruct((E_PAD, HID), jnp.float32),
    )(ea, wee, bee)


def _msg_body(hs, hd, e, w1s, w1d, w1e, b1, w2, b2, m_o):
    a = (_dot(hs[...], w1s[...]) + _dot(hd[...], w1d[...])
         + _dot(e[...], w1e[...]) + b1[...])
    m = _dot(jnp.maximum(a, 0.0), w2[...]) + b2[...]
    row = pl.program_id(0) * BE + lax.broadcasted_iota(jnp.int32, (BE, HID), 0)
    m_o[...] = jnp.where(row < E, m, 0.0)


def _msg(hs, hd, e, w1s, w1d, w1e, b1, w2, b2):
    hcol = hs.shape[1] // HID - 1
    hspec = pl.BlockSpec((BE, HID), lambda i: (i, hcol))
    return pl.pallas_call(
        _msg_body,
        grid=(GE,),
        in_specs=[
            hspec, hspec,
            pl.BlockSpec((BE, HID), lambda i: (i, 0)),
            _full((HID, 256)), _full((HID, 256)), _full((HID, 256)),
            _full((1, 256)), _full((256, HID)), _full((1, HID)),
        ],
        out_specs=pl.BlockSpec((BE, HID), lambda i: (i, 0)),
        out_shape=jax.ShapeDtypeStruct((E_PAD, HID), jnp.float32),
    )(hs, hd, e, w1s, w1d, w1e, b1, w2, b2)


def _make_node_upd(append_h):
    ow = 2 * HID if append_h else HID

    def body(h, agg2, wn1, wn2, bn, wa, wb, h_o, oa, ob):
        agg = agg2[0] + agg2[1]
        hp = jnp.maximum(
            _dot(h[...], wn1[...]) + _dot(agg, wn2[...]) + bn[...], 0.0)
        h_o[...] = hp
        for w, o in ((wa, oa), (wb, ob)):
            p = _dot(hp, w[...])
            o[...] = jnp.concatenate([p, hp], axis=-1) if append_h else p

    def call(h, agg2, wn1, wn2, bn, wa, wb):
        return pl.pallas_call(
            body,
            grid=(GN,),
            in_specs=[
                pl.BlockSpec((BN, HID), lambda i: (i, 0)),
                pl.BlockSpec((2, BN, HID), lambda i: (0, i, 0)),
                _full((HID, HID)), _full((HID, HID)), _full((1, HID)),
                _full((HID, HID)), _full((HID, HID)),
            ],
            out_specs=[pl.BlockSpec((BN, HID), lambda i: (i, 0))]
            + [pl.BlockSpec((BN, ow), lambda i: (i, 0))] * 2,
            out_shape=[jax.ShapeDtypeStruct((N, HID), jnp.float32)]
            + [jax.ShapeDtypeStruct((N, ow), jnp.float32)] * 2,
        )(h, agg2, wn1, wn2, bn, wa, wb)

    return call


_node_upd_mid = _make_node_upd(True)
_node_upd_last = _make_node_upd(False)


def _edge_upd_body(g3, g4, e, wec, bec, e_o):
    e_o[...] = jnp.maximum(
        g3[...] + g4[...] + _dot(e[...], wec[...]) + bec[...], 0.0)


def _edge_upd(g3, g4, e, wec, bec):
    gspec = pl.BlockSpec((BE, HID), lambda i: (i, 0))
    return pl.pallas_call(
        _edge_upd_body,
        grid=(GE,),
        in_specs=[
            gspec, gspec,
            pl.BlockSpec((BE, HID), lambda i: (i, 0)),
            _full((HID, HID)), _full((1, HID)),
        ],
        out_specs=pl.BlockSpec((BE, HID), lambda i: (i, 0)),
        out_shape=jax.ShapeDtypeStruct((E_PAD, HID), jnp.float32),
    )(g3, g4, e, wec, bec)


def _cls_body(u1, u2, ea, wce, bc1, gsc, beta, wc2, bc2, wc3, bc3, o):
    x = jnp.maximum(u1[...] + u2[...] + _dot(ea[...], wce[...]) + bc1[...], 0.0)
    x = x * gsc[...] + beta[...]
    x = jnp.maximum(_dot(x, wc2[...]) + bc2[...], 0.0)
    o[...] = _dot(x, wc3[...]) + bc3[...]


def _cls(u1, u2, ea, wce, bc1, gsc, beta, wc2, bc2, wc3, bc3):
    return pl.pallas_call(
        _cls_body,
        grid=(GE,),
        in_specs=[
            pl.BlockSpec((BE, HID), lambda i: (i, 0)),
            pl.BlockSpec((BE, HID), lambda i: (i, 0)),
            pl.BlockSpec((BE, 16), lambda i: (i, 0)),
            _full((16, HID)), _full((1, HID)),
            _full((1, HID)), _full((1, HID)),
            _full((HID, 64)), _full((1, 64)),
            _full((64, 8)), _full((1, 8)),
        ],
        out_specs=pl.BlockSpec((BE, 8), lambda i: (i, 0)),
        out_shape=jax.ShapeDtypeStruct((E_PAD, 8), jnp.float32),
    )(u1, u2, ea, wce, bc1, gsc, beta, wc2, bc2, wc3, bc3)


def kernel(node_features, edge_index, edge_attr, params):
    f32 = jnp.float32
    src3 = jnp.pad(edge_index[0], (0, E_PAD - E)).reshape(NW, NCHUNK, CHUNK)
    dst3 = jnp.pad(edge_index[1], (0, E_PAD - E)).reshape(NW, NCHUNK, CHUNK)
    ea = jnp.pad(edge_attr, ((0, E_PAD - E), (0, 0)))
    zeros = jnp.zeros((N_PAD, HID), f32)

    def row(b):
        return b.reshape(1, -1)

    lyr = params["layers"]
    w1s = [lp["msg1"]["w"][:HID] for lp in lyr]
    w1d = [lp["msg1"]["w"][HID:2 * HID] for lp in lyr]
    w1e = [lp["msg1"]["w"][2 * HID:] for lp in lyr]
    b1 = [row(lp["msg1"]["b"]) for lp in lyr]
    wn1 = [lp["node_upd"]["w"][:HID] for lp in lyr]
    wn2 = [lp["node_upd"]["w"][HID:] for lp in lyr]
    bn = [row(lp["node_upd"]["b"]) for lp in lyr]
    wes = [lp["edge_upd"]["w"][:HID] for lp in lyr]
    wed = [lp["edge_upd"]["w"][HID:2 * HID] for lp in lyr]
    wec = [lp["edge_upd"]["w"][2 * HID:] for lp in lyr]
    bec = [row(lp["edge_upd"]["b"]) for lp in lyr]
    w2 = [lp["msg2"]["w"] for lp in lyr]
    b2 = [row(lp["msg2"]["b"]) for lp in lyr]
    wcs = params["cls1"]["w"][:HID]
    wcd = params["cls1"]["w"][HID:2 * HID]
    wce = params["cls1"]["w"][2 * HID:]
    bc1 = row(params["cls1"]["b"])
    inv = 1.0 / jnp.sqrt(jnp.asarray(1.0 + 1e-5, f32))
    gsc = row(params["bn_gamma"] * inv)
    beta = row(params["bn_beta"])
    wc2 = params["cls2"]["w"]
    bc2 = row(params["cls2"]["b"])
    wc3 = jnp.pad(params["cls3"]["w"], ((0, 0), (0, 6)))
    bc3 = jnp.pad(row(params["cls3"]["b"]), ((0, 0), (0, 6)))

    h = _node_enc(node_features, params["node_enc"]["w"],
                  row(params["node_enc"]["b"]))
    e = _edge_enc(ea, params["edge_enc"]["w"], row(params["edge_enc"]["b"]))

    hs, hd = _gather2_128(h, h, src3, dst3)
    for l in range(3):
        m = _msg(hs, hd, e, w1s[l], w1d[l], w1e[l], b1[l], w2[l], b2[l])
        agg2 = _scatter_add(m, dst3, zeros)
        if l < 2:
            h, rh, sh = _node_upd_mid(h, agg2, wn1[l], wn2[l], bn[l],
                                      wes[l], wed[l])
            hs, hd = _gather2_256(rh, sh, src3, dst3)
            e = _edge_upd(hs, hd, e, wec[l], bec[l])
        else:
            h, t1, t2 = _node_upd_last(h, agg2, wn1[l], wn2[l], bn[l],
                                       wcs, wcd)

    u1, u2 = _gather2_128(t1, t2, src3, dst3)
    logits = _cls(u1, u2, ea, wce, bc1, gsc, beta, wc2, bc2, wc3, bc3)
    return logits[:E, :2]


# speedup vs baseline: 1.7783x; 1.1211x over previous
import functools

import jax
import jax.numpy as jnp
from jax import lax
from jax.experimental import pallas as pl
from jax.experimental.pallas import tpu as pltpu
from jax.experimental.pallas import tpu_sc as plsc

N = 10000
E = 160000
HID = 128
E_PAD = 163840
NW = 32
PER_W = E_PAD // NW
CHUNK = 128
NCHUNK = PER_W // CHUNK
BE = 2048
GE = E_PAD // BE
BN = 2000
GN = N // BN
N_PAD = 10240
ROWS_PER_TILE = N_PAD // 16


def _sc_mesh():
    return plsc.VectorSubcoreMesh(
        core_axis_name="c", subcore_axis_name="s", num_cores=2, num_subcores=16
    )


@functools.lru_cache(maxsize=None)
def _make_gather2(D, dtype):
    NB = min(8, (131071 - 2 * NCHUNK * CHUNK) // (CHUNK * D))

    @functools.partial(
        pl.kernel,
        mesh=_sc_mesh(),
        out_type=[
            jax.ShapeDtypeStruct((E_PAD, D), dtype),
            jax.ShapeDtypeStruct((E_PAD, D), dtype),
        ],
        scratch_types=[
            pltpu.VMEM((NCHUNK, CHUNK), jnp.int32),
            pltpu.VMEM((NCHUNK, CHUNK), jnp.int32),
        ]
        + [pltpu.VMEM((CHUNK, D), dtype) for _ in range(NB)]
        + [pltpu.SemaphoreType.DMA for _ in range(2 * NB)],
    )
    def gk(t1, t2, i1, i2, o1, o2, i1_v, i2_v, *rest):
        bufs = rest[0:NB]
        gsem = rest[NB:2 * NB]
        wsem = rest[2 * NB:3 * NB]
        wid = lax.axis_index("s") * 2 + lax.axis_index("c")
        base = wid * PER_W
        pltpu.sync_copy(i1.at[wid], i1_v)
        pltpu.sync_copy(i2.at[wid], i2_v)
        ops = [(t, j) for j in range(NCHUNK) for t in (0, 1)]

        def start(k, s):
            t, j = ops[k]
            idx = (i1_v if t == 0 else i2_v).at[j]
            tab = t1 if t == 0 else t2
            return pltpu.async_copy(tab.at[idx], bufs[s], gsem[s])

        def wout(k, s):
            t, j = ops[k]
            out = o1 if t == 0 else o2
            return pltpu.async_copy(
                bufs[s], out.at[pl.ds(base + j * CHUNK, CHUNK)], wsem[s])

        g = [None] * NB
        w = [None] * NB
        LAG = max(1, NB // 2)
        for k in range(len(ops)):
            s = k % NB
            if w[s] is not None:
                w[s].wait()
            g[s] = start(k, s)
            ko = k - LAG
            if ko >= 0:
                so = ko % NB
                g[so].wait()
                w[so] = wout(ko, so)
        for ko in range(len(ops) - LAG, len(ops)):
            so = ko % NB
            g[so].wait()
            w[so] = wout(ko, so)
        for ko in range(len(ops) - NB, len(ops)):
            w[ko % NB].wait()

    return gk


def _gather2_f32(t1, t2, i1, i2):
    return _make_gather2(128, jnp.float32)(t1, t2, i1, i2)


def _gather2_packed(t1, t2, i1, i2):
    return _make_gather2(128, jnp.int32)(t1, t2, i1, i2)


KS = 2


@functools.lru_cache(maxsize=None)
def _make_scatter_add():
    @functools.partial(
        pl.kernel,
        mesh=_sc_mesh(),
        out_type=jax.ShapeDtypeStruct((2, N_PAD, HID), jnp.float32),
        scratch_types=[
            pltpu.VMEM((NCHUNK, CHUNK), jnp.int32),
            pltpu.VMEM_SHARED((N_PAD, HID), jnp.float32),
        ]
        + [pltpu.VMEM((CHUNK, HID), jnp.float32) for _ in range(KS)]
        + [pltpu.SemaphoreType.DMA for _ in range(2 * KS)],
    )
    def sk(m, dst3, zeros, out, idx_v, acc, *rest):
        rows = rest[0:KS]
        ls = rest[KS:2 * KS]
        ss = rest[2 * KS:3 * KS]
        cid = lax.axis_index("c")
        sid = lax.axis_index("s")
        wid = sid * 2 + cid
        base = wid * PER_W
        r0 = sid * ROWS_PER_TILE
        pltpu.sync_copy(dst3.at[wid], idx_v)
        ld = [None] * KS
        sc = [None] * KS
        for j in range(KS):
            ld[j] = pltpu.async_copy(
                m.at[pl.ds(base + j * CHUNK, CHUNK)], rows[j], ls[j])
        pltpu.sync_copy(zeros.at[pl.ds(r0, ROWS_PER_TILE)],
                        acc.at[pl.ds(r0, ROWS_PER_TILE)])
        plsc.subcore_barrier()
        for j in range(NCHUNK):
            s = j % KS
            ld[s].wait()
            sc[s] = pltpu.async_copy(rows[s], acc.at[idx_v.at[j]], ss[s],
                                     add=True)
            nj = j + KS
            if nj < NCHUNK:
                sc[s].wait()
                ld[s] = pltpu.async_copy(
                    m.at[pl.ds(base + nj * CHUNK, CHUNK)], rows[s], ls[s])
        for j in range(NCHUNK - KS, NCHUNK):
            sc[j % KS].wait()
        plsc.subcore_barrier()
        pltpu.sync_copy(acc.at[pl.ds(r0, ROWS_PER_TILE)],
                        out.at[cid, pl.ds(r0, ROWS_PER_TILE)])

    return sk


def _scatter_add(m, dst3, zeros):
    return _make_scatter_add()(m, dst3, zeros)


def _dot(a, b):
    return jnp.dot(a, b, preferred_element_type=jnp.float32)


def _pack_bf16(x):
    c = x.shape[-1] // 2
    u = jnp.uint32
    lo = lax.bitcast_convert_type(
        x[:, :c].astype(jnp.bfloat16).astype(jnp.float32), u)
    hi = lax.bitcast_convert_type(
        x[:, c:].astype(jnp.bfloat16).astype(jnp.float32), u)
    return lax.bitcast_convert_type(hi | (lo >> 16), jnp.int32)


def _unpack_bf16(p):
    u = lax.bitcast_convert_type(p, jnp.uint32)
    lo = lax.bitcast_convert_type(u << 16, jnp.float32)
    hi = lax.bitcast_convert_type(u & jnp.uint32(0xFFFF0000), jnp.float32)
    return jnp.concatenate([lo, hi], axis=-1)


def _full(shape):
    return pl.BlockSpec(shape, lambda i: tuple(0 for _ in shape))


def _node_enc_body(nf, wne, bne, h_o):
    h_o[...] = _dot(nf[...], wne[...]) + bne[...]


def _node_enc(nf, wne, bne):
    return pl.pallas_call(
        _node_enc_body,
        grid=(GN,),
        in_specs=[
            pl.BlockSpec((BN, HID), lambda i: (i, 0)),
            _full((HID, HID)), _full((1, HID)),
        ],
        out_specs=pl.BlockSpec((BN, HID), lambda i: (i, 0)),
        out_shape=jax.ShapeDtypeStruct((N, HID), jnp.float32),
    )(nf, wne, bne)


def _edge_enc_body(ea, wee, bee, e_o):
    e_o[...] = _dot(ea[...], wee[...]) + bee[...]


def _edge_enc(ea, wee, bee):
    return pl.pallas_call(
        _edge_enc_body,
        grid=(GE,),
        in_specs=[
            pl.BlockSpec((BE, 16), lambda i: (i, 0)),
            _full((16, HID)), _full((1, HID)),
        ],
        out_specs=pl.BlockSpec((BE, HID), lambda i: (i, 0)),
        out_shape=jax.ShapeDtypeStruct((E_PAD, HID), jnp.float32),
    )(ea, wee, bee)


def _msg(hs, hd, e, w1s, w1d, w1e, b1, w2, b2):
    packed = hs.dtype == jnp.int32

    def body(hs, hd, e, w1s, w1d, w1e, b1, w2, b2, m_o):
        bf = jnp.bfloat16
        if packed:
            hsv = _unpack_bf16(hs[...][:, 64:]).astype(bf)
            hdv = _unpack_bf16(hd[...][:, 64:]).astype(bf)
        else:
            hsv = hs[...].astype(bf)
            hdv = hd[...].astype(bf)
        a = (_dot(hsv, w1s[...].astype(bf)) + _dot(hdv, w1d[...].astype(bf))
             + _dot(e[...], w1e[...]) + b1[...])
        m = _dot(jnp.maximum(a, 0.0), w2[...]) + b2[...]
        row = pl.program_id(0) * BE + lax.broadcasted_iota(
            jnp.int32, (BE, HID), 0)
        m_o[...] = jnp.where(row < E, m, 0.0)

    hspec = pl.BlockSpec((BE, HID), lambda i: (i, 0))
    return pl.pallas_call(
        body,
        grid=(GE,),
        in_specs=[
            hspec, hspec,
            pl.BlockSpec((BE, HID), lambda i: (i, 0)),
            _full((HID, 256)), _full((HID, 256)), _full((HID, 256)),
            _full((1, 256)), _full((256, HID)), _full((1, HID)),
        ],
        out_specs=pl.BlockSpec((BE, HID), lambda i: (i, 0)),
        out_shape=jax.ShapeDtypeStruct((E_PAD, HID), jnp.float32),
    )(hs, hd, e, w1s, w1d, w1e, b1, w2, b2)


def _make_node_upd(append_h):
    dt = jnp.int32 if append_h else jnp.float32

    def body(h, agg2, wn1, wn2, bn, wa, wb, h_o, oa, ob):
        agg = agg2[0] + agg2[1]
        hp = jnp.maximum(
            _dot(h[...], wn1[...]) + _dot(agg, wn2[...]) + bn[...], 0.0)
        h_o[...] = hp
        hpk = _pack_bf16(hp) if append_h else None
        for w, o in ((wa, oa), (wb, ob)):
            p = _dot(hp, w[...])
            if append_h:
                o[...] = jnp.concatenate([_pack_bf16(p), hpk], axis=-1)
            else:
                o[...] = p

    def call(h, agg2, wn1, wn2, bn, wa, wb):
        return pl.pallas_call(
            body,
            grid=(GN,),
            in_specs=[
                pl.BlockSpec((BN, HID), lambda i: (i, 0)),
                pl.BlockSpec((2, BN, HID), lambda i: (0, i, 0)),
                _full((HID, HID)), _full((HID, HID)), _full((1, HID)),
                _full((HID, HID)), _full((HID, HID)),
            ],
            out_specs=[pl.BlockSpec((BN, HID), lambda i: (i, 0))] * 3,
            out_shape=[jax.ShapeDtypeStruct((N, HID), jnp.float32)]
            + [jax.ShapeDtypeStruct((N, HID), dt)] * 2,
        )(h, agg2, wn1, wn2, bn, wa, wb)

    return call


_node_upd_mid = _make_node_upd(True)
_node_upd_last = _make_node_upd(False)


def _edge_upd_body(g3, g4, e, wec, bec, e_o):
    e_o[...] = jnp.maximum(
        _unpack_bf16(g3[...][:, :64]) + _unpack_bf16(g4[...][:, :64])
        + _dot(e[...], wec[...]) + bec[...], 0.0)


def _edge_upd(g3, g4, e, wec, bec):
    gspec = pl.BlockSpec((BE, HID), lambda i: (i, 0))
    return pl.pallas_call(
        _edge_upd_body,
        grid=(GE,),
        in_specs=[
            gspec, gspec,
            pl.BlockSpec((BE, HID), lambda i: (i, 0)),
            _full((HID, HID)), _full((1, HID)),
        ],
        out_specs=pl.BlockSpec((BE, HID), lambda i: (i, 0)),
        out_shape=jax.ShapeDtypeStruct((E_PAD, HID), jnp.float32),
    )(g3, g4, e, wec, bec)


def _cls_body(u1, u2, ea, wce, bc1, gsc, beta, wc2, bc2, wc3, bc3, o):
    x = jnp.maximum(u1[...] + u2[...]
                    + _dot(ea[...], wce[...]) + bc1[...], 0.0)
    x = x * gsc[...] + beta[...]
    x = jnp.maximum(_dot(x, wc2[...]) + bc2[...], 0.0)
    o[...] = _dot(x, wc3[...]) + bc3[...]


def _cls(u1, u2, ea, wce, bc1, gsc, beta, wc2, bc2, wc3, bc3):
    return pl.pallas_call(
        _cls_body,
        grid=(GE,),
        in_specs=[
            pl.BlockSpec((BE, HID), lambda i: (i, 0)),
            pl.BlockSpec((BE, HID), lambda i: (i, 0)),
            pl.BlockSpec((BE, 16), lambda i: (i, 0)),
            _full((16, HID)), _full((1, HID)),
            _full((1, HID)), _full((1, HID)),
            _full((HID, 64)), _full((1, 64)),
            _full((64, 8)), _full((1, 8)),
        ],
        out_specs=pl.BlockSpec((BE, 8), lambda i: (i, 0)),
        out_shape=jax.ShapeDtypeStruct((E_PAD, 8), jnp.float32),
    )(u1, u2, ea, wce, bc1, gsc, beta, wc2, bc2, wc3, bc3)


def kernel(node_features, edge_index, edge_attr, params):
    f32 = jnp.float32
    src3 = jnp.pad(edge_index[0], (0, E_PAD - E)).reshape(NW, NCHUNK, CHUNK)
    dst3 = jnp.pad(edge_index[1], (0, E_PAD - E)).reshape(NW, NCHUNK, CHUNK)
    ea = jnp.pad(edge_attr, ((0, E_PAD - E), (0, 0)))
    zeros = jnp.zeros((N_PAD, HID), f32)

    def row(b):
        return b.reshape(1, -1)

    lyr = params["layers"]
    w1s = [lp["msg1"]["w"][:HID] for lp in lyr]
    w1d = [lp["msg1"]["w"][HID:2 * HID] for lp in lyr]
    w1e = [lp["msg1"]["w"][2 * HID:] for lp in lyr]
    b1 = [row(lp["msg1"]["b"]) for lp in lyr]
    wn1 = [lp["node_upd"]["w"][:HID] for lp in lyr]
    wn2 = [lp["node_upd"]["w"][HID:] for lp in lyr]
    bn = [row(lp["node_upd"]["b"]) for lp in lyr]
    wes = [lp["edge_upd"]["w"][:HID] for lp in lyr]
    wed = [lp["edge_upd"]["w"][HID:2 * HID] for lp in lyr]
    wec = [lp["edge_upd"]["w"][2 * HID:] for lp in lyr]
    bec = [row(lp["edge_upd"]["b"]) for lp in lyr]
    w2 = [lp["msg2"]["w"] for lp in lyr]
    b2 = [row(lp["msg2"]["b"]) for lp in lyr]
    wcs = params["cls1"]["w"][:HID]
    wcd = params["cls1"]["w"][HID:2 * HID]
    wce = params["cls1"]["w"][2 * HID:]
    bc1 = row(params["cls1"]["b"])
    inv = 1.0 / jnp.sqrt(jnp.asarray(1.0 + 1e-5, f32))
    gsc = row(params["bn_gamma"] * inv)
    beta = row(params["bn_beta"])
    wc2 = params["cls2"]["w"]
    bc2 = row(params["cls2"]["b"])
    wc3 = jnp.pad(params["cls3"]["w"], ((0, 0), (0, 6)))
    bc3 = jnp.pad(row(params["cls3"]["b"]), ((0, 0), (0, 6)))

    h = _node_enc(node_features, params["node_enc"]["w"],
                  row(params["node_enc"]["b"]))
    e = _edge_enc(ea, params["edge_enc"]["w"], row(params["edge_enc"]["b"]))

    hs, hd = _gather2_f32(h, h, src3, dst3)
    for l in range(3):
        m = _msg(hs, hd, e, w1s[l], w1d[l], w1e[l], b1[l], w2[l], b2[l])
        agg2 = _scatter_add(m, dst3, zeros)
        if l < 2:
            h, rh, sh = _node_upd_mid(h, agg2, wn1[l], wn2[l], bn[l],
                                      wes[l], wed[l])
            hs, hd = _gather2_packed(rh, sh, src3, dst3)
            e = _edge_upd(hs, hd, e, wec[l], bec[l])
        else:
            h, t1, t2 = _node_upd_last(h, agg2, wn1[l], wn2[l], bn[l],
                                       wcs, wcd)

    u1, u2 = _gather2_f32(t1, t2, src3, dst3)
    logits = _cls(u1, u2, ea, wce, bc1, gsc, beta, wc2, bc2, wc3, bc3)
    return logits[:E, :2]
